# Initial kernel scaffold; baseline (speedup 1.0000x reference)
import jax
import jax.numpy as jnp
from jax.experimental import pallas as pl


def kernel(kv_cache, tgt_loc, src_loc):
    raise NotImplementedError("write your pallas kernel here")


# speedup vs baseline: 16.1967x; 16.1967x over previous
import functools

import jax
import jax.numpy as jnp
from jax import lax
from jax.experimental import pallas as pl
from jax.experimental.pallas import tpu as pltpu
from jax.experimental.pallas import tpu_sc as plsc

_NUM_LAYERS = 16
_NUM_SLOTS = 4096
_ROW_DIM = 512
_NUM_LOCS = 4096

_NC = 2
_NS = 16
_NW = _NC * _NS
_SLOTS_PER_TILE = (_NUM_LAYERS * _NUM_SLOTS) // _NW
_CHUNK = 64


def _body(cache_hbm, tgt_hbm, src_hbm, out_hbm,
          tgt_v, src_v, final_v, idx_v, buf_v, sem):
    wid = lax.axis_index("s") * _NC + lax.axis_index("c")

    pltpu.sync_copy(tgt_hbm, tgt_v)
    pltpu.sync_copy(src_hbm, src_v)

    lane = lax.iota(jnp.int32, 16)

    def init_body(j, _):
        final_v[pl.ds(j * 16, 16)] = lane + j * 16
        return 0

    lax.fori_loop(0, _NUM_SLOTS // 16, init_body, 0)

    next_lane = jnp.minimum(lane + 1, 15)

    def scatter_body(i, _):
        t = tgt_v[pl.ds(i * 16, 16)]
        s = src_v[pl.ds(i * 16, 16)]
        key_sorted, s_sorted = plsc.sort_key_val(t * 16 + lane, s)
        t_sorted = key_sorted >> 4
        t_next = t_sorted.at[next_lane].get(mode="promise_in_bounds")
        is_last = (t_sorted != t_next) | (lane == 15)
        plsc.store_scatter(final_v, [t_sorted], s_sorted, mask=is_last)
        return 0

    lax.fori_loop(0, _NUM_LOCS // 16, scatter_body, 0)

    layer = wid // 2
    half = wid % 2
    layer_off = layer * _NUM_SLOTS
    t0_base = half * (_NUM_SLOTS // 2)

    def chunk_body(cidx, _):
        t0 = t0_base + cidx * _CHUNK

        def idx_body(j, _):
            idx_v[pl.ds(j * 16, 16)] = (
                final_v[pl.ds(t0 + j * 16, 16)] + layer_off)
            return 0

        lax.fori_loop(0, _CHUNK // 16, idx_body, 0)
        pltpu.async_copy(cache_hbm.at[idx_v], buf_v, sem).wait()
        pltpu.sync_copy(buf_v, out_hbm.at[pl.ds(layer_off + t0, _CHUNK)])
        return 0

    lax.fori_loop(0, (_NUM_SLOTS // 2) // _CHUNK, chunk_body, 0)


@jax.jit
def _sc_copy(cache2d, tgt32, src32):
    mesh = plsc.VectorSubcoreMesh(core_axis_name="c", subcore_axis_name="s")
    return pl.kernel(
        _body,
        out_type=jax.ShapeDtypeStruct(
            (_NUM_LAYERS * _NUM_SLOTS, _ROW_DIM), jnp.float32),
        mesh=mesh,
        compiler_params=pltpu.CompilerParams(needs_layout_passes=False),
        scratch_types=[
            pltpu.VMEM((_NUM_LOCS,), jnp.int32),
            pltpu.VMEM((_NUM_LOCS,), jnp.int32),
            pltpu.VMEM((_NUM_SLOTS,), jnp.int32),
            pltpu.VMEM((_CHUNK,), jnp.int32),
            pltpu.VMEM((_CHUNK, _ROW_DIM), jnp.float32),
            pltpu.SemaphoreType.DMA,
        ],
    )(cache2d, tgt32, src32)


def kernel(kv_cache, tgt_loc, src_loc):
    cache2d = kv_cache.reshape(_NUM_LAYERS * _NUM_SLOTS, _ROW_DIM)
    out2d = _sc_copy(cache2d,
                     tgt_loc.astype(jnp.int32),
                     src_loc.astype(jnp.int32))
    return out2d.reshape(kv_cache.shape)


# speedup vs baseline: 18.7996x; 1.1607x over previous
import functools

import jax
import jax.numpy as jnp
from jax import lax
from jax.experimental import pallas as pl
from jax.experimental.pallas import tpu as pltpu
from jax.experimental.pallas import tpu_sc as plsc

_NUM_LAYERS = 16
_NUM_SLOTS = 4096
_ROW_DIM = 512
_NUM_LOCS = 4096

_NC = 2
_NS = 16
_NW = _NC * _NS
_SLOTS_PER_TILE = (_NUM_LAYERS * _NUM_SLOTS) // _NW
_CHUNK = 32
_NBUF = 4


def _body(cache_hbm, tgt_hbm, src_hbm, out_hbm,
          tgt_v, src_v, final_v, idx_vs, buf_vs, gsems, ssems):
    wid = lax.axis_index("s") * _NC + lax.axis_index("c")

    pltpu.sync_copy(tgt_hbm, tgt_v)
    pltpu.sync_copy(src_hbm, src_v)

    lane = lax.iota(jnp.int32, 16)

    def init_body(j, _):
        final_v[pl.ds(j * 16, 16)] = lane + j * 16
        return 0

    lax.fori_loop(0, _NUM_SLOTS // 16, init_body, 0)

    next_lane = jnp.minimum(lane + 1, 15)

    def scatter_body(i, _):
        t = tgt_v[pl.ds(i * 16, 16)]
        s = src_v[pl.ds(i * 16, 16)]
        key_sorted, s_sorted = plsc.sort_key_val(t * 16 + lane, s)
        t_sorted = key_sorted >> 4
        t_next = t_sorted.at[next_lane].get(mode="promise_in_bounds")
        is_last = (t_sorted != t_next) | (lane == 15)
        plsc.store_scatter(final_v, [t_sorted], s_sorted, mask=is_last)
        return 0

    lax.fori_loop(0, _NUM_LOCS // 16, scatter_body, 0)

    layer = wid // 2
    half = wid % 2
    layer_off = layer * _NUM_SLOTS
    t0_base = half * (_NUM_SLOTS // 2)

    def build_idx(idx_v, t0):
        def idx_body(j, _):
            idx_v[pl.ds(j * 16, 16)] = (
                final_v[pl.ds(t0 + j * 16, 16)] + layer_off)
            return 0

        lax.fori_loop(0, _CHUNK // 16, idx_body, 0)

    def group_body(g, _):
        for b in range(_NBUF):
            t0 = t0_base + (g * _NBUF + b) * _CHUNK

            @pl.when(g > 0)
            def _wait_prev_store():
                pltpu.make_async_copy(
                    buf_vs[b], out_hbm.at[pl.ds(layer_off + t0, _CHUNK)],
                    ssems[b]).wait()

            build_idx(idx_vs[b], t0)
            pltpu.async_copy(cache_hbm.at[idx_vs[b]], buf_vs[b], gsems[b])
        for b in range(_NBUF):
            t0 = t0_base + (g * _NBUF + b) * _CHUNK
            pltpu.make_async_copy(cache_hbm.at[idx_vs[b]], buf_vs[b],
                                  gsems[b]).wait()
            pltpu.async_copy(buf_vs[b],
                             out_hbm.at[pl.ds(layer_off + t0, _CHUNK)],
                             ssems[b])
        return 0

    ngroups = (_NUM_SLOTS // 2) // (_CHUNK * _NBUF)
    lax.fori_loop(0, ngroups, group_body, 0)
    for b in range(_NBUF):
        t0 = t0_base + ((ngroups - 1) * _NBUF + b) * _CHUNK
        pltpu.make_async_copy(
            buf_vs[b], out_hbm.at[pl.ds(layer_off + t0, _CHUNK)],
            ssems[b]).wait()


@jax.jit
def _sc_copy(cache2d, tgt32, src32):
    mesh = plsc.VectorSubcoreMesh(core_axis_name="c", subcore_axis_name="s")
    return pl.kernel(
        _body,
        out_type=jax.ShapeDtypeStruct(
            (_NUM_LAYERS * _NUM_SLOTS, _ROW_DIM), jnp.float32),
        mesh=mesh,
        compiler_params=pltpu.CompilerParams(needs_layout_passes=False),
        scratch_types=[
            pltpu.VMEM((_NUM_LOCS,), jnp.int32),
            pltpu.VMEM((_NUM_LOCS,), jnp.int32),
            pltpu.VMEM((_NUM_SLOTS,), jnp.int32),
            [pltpu.VMEM((_CHUNK,), jnp.int32) for _ in range(_NBUF)],
            [pltpu.VMEM((_CHUNK, _ROW_DIM), jnp.float32)
             for _ in range(_NBUF)],
            [pltpu.SemaphoreType.DMA for _ in range(_NBUF)],
            [pltpu.SemaphoreType.DMA for _ in range(_NBUF)],
        ],
    )(cache2d, tgt32, src32)


def kernel(kv_cache, tgt_loc, src_loc):
    cache2d = kv_cache.reshape(_NUM_LAYERS * _NUM_SLOTS, _ROW_DIM)
    out2d = _sc_copy(cache2d,
                     tgt_loc.astype(jnp.int32),
                     src_loc.astype(jnp.int32))
    return out2d.reshape(kv_cache.shape)
